# Initial kernel scaffold; baseline (speedup 1.0000x reference)
import functools

import jax
import jax.numpy as jnp
from jax import lax
from jax.experimental import pallas as pl
from jax.experimental.pallas import tpu as pltpu
from jax.experimental.pallas import tpu_sc as plsc

N_NODE = 100000
D = 32
E = 1600000
HALF = 50000
ACC_ROWS = 50176
CROWS = 3136
DUMMY = HALF
EPT = E // 16
BLK = 2000
NBLK = EPT // BLK
CH = 80
NCH = BLK // CH


def _sc_body(edge, user_t, item_t, zrow, zcnt,
             sum_i, cnt_i, sum_u, cnt_u,
             gbuf, dbuf, sidx, rows, cntl, idxb, acc, cnts, sem):
    c = lax.axis_index("c")
    s = lax.axis_index("s")
    lo = c * HALF
    ones16 = jnp.ones((16,), jnp.float32)
    iota16 = lax.iota(jnp.int32, 16)

    def zero_acc():
        pltpu.sync_copy(zrow, acc.at[pl.ds(s * 3136, 3136)])
        pltpu.sync_copy(zcnt, cntl)
        pltpu.sync_copy(zcnt.at[pl.ds(0, 196)], cnts.at[pl.ds(s * 196, 196)])

    def direction(g_row, s_row, table):
        @pl.loop(0, NBLK)
        def blk_body(blk):
            base = s * EPT + blk * BLK
            pltpu.sync_copy(edge.at[g_row, pl.ds(base, BLK)], gbuf)
            pltpu.sync_copy(edge.at[s_row, pl.ds(base, BLK)], dbuf)

            @pl.loop(0, NCH)
            def ch_body(ch):
                off = ch * CH
                for q in range(CH // 16):
                    dv = dbuf[pl.ds(off + q * 16, 16)]
                    ld = dv - lo
                    m = (ld >= 0) & (ld < HALF)
                    ld = jnp.where(m, ld, DUMMY)
                    sidx[pl.ds(q * 16, 16)] = ld
                    plsc.addupdate_scatter(
                        cntl, [ld >> 4, ld & 15], ones16, mask=m)
                pltpu.async_copy(
                    table.at[gbuf.at[pl.ds(off, CH)]], rows, sem).wait()
                pltpu.sync_copy(rows, acc.at[sidx], add=True)

    def merge_counts():
        @pl.loop(0, CROWS // 112)
        def mrg(j):
            for q in range(7):
                idxb[pl.ds(q * 16, 16)] = j * 112 + q * 16 + iota16
            pltpu.sync_copy(cntl.at[pl.ds(j * 112, 112)],
                            cnts.at[idxb], add=True)

    def write_out(sum_o, cnt_o):
        @pl.loop(0, 25)
        def wrs(q):
            lbase = s * 3125 + q * 125
            pltpu.sync_copy(acc.at[pl.ds(lbase, 125)],
                            sum_o.at[pl.ds(c * HALF + lbase, 125)])

        @pl.loop(0, 2)
        def wrc(k):
            j = k * 16 + s

            @pl.when(j < 25)
            def _():
                pltpu.sync_copy(cnts.at[pl.ds(j * 125, 125)],
                                cnt_o.at[pl.ds(c * 3125 + j * 125, 125)])

    zero_acc()
    plsc.subcore_barrier()
    direction(0, 1, user_t)
    plsc.subcore_barrier()
    merge_counts()
    plsc.subcore_barrier()
    write_out(sum_i, cnt_i)
    plsc.subcore_barrier()
    zero_acc()
    plsc.subcore_barrier()
    direction(1, 0, item_t)
    plsc.subcore_barrier()
    merge_counts()
    plsc.subcore_barrier()
    write_out(sum_u, cnt_u)


_sc_call = pl.kernel(
    _sc_body,
    out_type=(
        jax.ShapeDtypeStruct((N_NODE, D), jnp.float32),
        jax.ShapeDtypeStruct((6250, 16), jnp.float32),
        jax.ShapeDtypeStruct((N_NODE, D), jnp.float32),
        jax.ShapeDtypeStruct((6250, 16), jnp.float32),
    ),
    mesh=plsc.VectorSubcoreMesh(core_axis_name="c", subcore_axis_name="s"),
    scratch_types=(
        pltpu.VMEM((BLK,), jnp.int32),
        pltpu.VMEM((BLK,), jnp.int32),
        pltpu.VMEM((CH,), jnp.int32),
        pltpu.VMEM((CH, D), jnp.float32),
        pltpu.VMEM((CROWS, 16), jnp.float32),
        pltpu.VMEM((112,), jnp.int32),
        pltpu.VMEM_SHARED((ACC_ROWS, D), jnp.float32),
        pltpu.VMEM_SHARED((CROWS, 16), jnp.float32),
        pltpu.SemaphoreType.DMA,
    ),
)

BR = 1000


def _dense_body(hu, su, cu, hi, si, ci, wsT, wnT, bb, ou, oi):
    ws = wsT[...]
    wn = wnT[...]
    bv = bb[...]
    agg_u = su[...] * (1.0 / jnp.maximum(cu[...], 1.0))
    ou[...] = (jnp.dot(hu[...], ws, preferred_element_type=jnp.float32)
               + jnp.dot(agg_u, wn, preferred_element_type=jnp.float32) + bv)
    agg_i = si[...] * (1.0 / jnp.maximum(ci[...], 1.0))
    oi[...] = (jnp.dot(hi[...], ws, preferred_element_type=jnp.float32)
               + jnp.dot(agg_i, wn, preferred_element_type=jnp.float32) + bv)


def _row_spec():
    return pl.BlockSpec((BR, D), lambda i: (i, 0))


def _cnt_spec():
    return pl.BlockSpec((BR, 1), lambda i: (i, 0))


def _full_spec(shape):
    return pl.BlockSpec(shape, lambda i: tuple(0 for _ in shape))


_dense_call = pl.pallas_call(
    _dense_body,
    grid=(N_NODE // BR,),
    in_specs=[
        _row_spec(), _row_spec(), _cnt_spec(),
        _row_spec(), _row_spec(), _cnt_spec(),
        _full_spec((D, D)), _full_spec((D, D)), _full_spec((1, D)),
    ],
    out_specs=[_row_spec(), _row_spec()],
    out_shape=(
        jax.ShapeDtypeStruct((N_NODE, D), jnp.float32),
        jax.ShapeDtypeStruct((N_NODE, D), jnp.float32),
    ),
)


def kernel(edge_index, user_table, item_table, W_self, W_neigh, b):
    edge_index = edge_index.astype(jnp.int32)
    zrow = jnp.zeros((3136, D), jnp.float32)
    zcnt = jnp.zeros((3136, 16), jnp.float32)
    sum_i, cnt_i, sum_u, cnt_u = _sc_call(
        edge_index, user_table, item_table, zrow, zcnt)
    cnt_i = cnt_i.reshape(N_NODE, 1)
    cnt_u = cnt_u.reshape(N_NODE, 1)
    out_user, out_item = _dense_call(
        user_table, sum_u, cnt_u, item_table, sum_i, cnt_i,
        W_self.T, W_neigh.T, b.reshape(1, D))
    return (out_user, out_item)


# speedup vs baseline: 3.4275x; 3.4275x over previous
import jax
import jax.numpy as jnp
from jax import lax
from jax.experimental import pallas as pl
from jax.experimental.pallas import tpu as pltpu
from jax.experimental.pallas import tpu_sc as plsc

N_NODE = 100000
D = 32
E = 1600000
HALF = 50000
ACC_ROWS = 50176
DUMMY = HALF
EPT = E // 16
BLK = 2000
NBLK = EPT // BLK
CH = 80
NCH = BLK // CH


def _sc_body(src_e, dst_e, user_t, item_t, zrow, ones_r,
             sum_i, cnt_i, sum_u, cnt_u,
             gbuf, dbuf, sidx, rows, ones_v, acc, sem):
    c = lax.axis_index("c")
    s = lax.axis_index("s")
    lo = c * HALF

    def zero_acc():
        pltpu.sync_copy(zrow, acc.at[pl.ds(s * 3136, 3136)])

    def scan_pass(s_arr, g_arr, table):
        @pl.loop(0, NBLK)
        def blk_body(blk):
            base = s * EPT + blk * BLK
            if table is not None:
                pltpu.sync_copy(g_arr.at[pl.ds(base, BLK)], gbuf)
            pltpu.sync_copy(s_arr.at[pl.ds(base, BLK)], dbuf)

            @pl.loop(0, NCH)
            def ch_body(ch):
                off = ch * CH
                for q in range(CH // 16):
                    dv = dbuf[pl.ds(off + q * 16, 16)]
                    ld = dv - lo
                    m = (ld >= 0) & (ld < HALF)
                    ld = jnp.where(m, ld, DUMMY)
                    sidx[pl.ds(q * 16, 16)] = ld
                if table is not None:
                    pltpu.async_copy(
                        table.at[gbuf.at[pl.ds(off, CH)]], rows, sem).wait()
                    pltpu.sync_copy(rows, acc.at[sidx], add=True)
                else:
                    pltpu.sync_copy(ones_v, acc.at[sidx], add=True)

    def write_out(out_hbm):
        @pl.loop(0, 40)
        def wrs(k):
            j = k * 16 + s

            @pl.when(j < HALF // 80)
            def _():
                pltpu.sync_copy(acc.at[pl.ds(j * 80, 80)],
                                out_hbm.at[pl.ds(c * HALF + j * 80, 80)])

    def phase(s_arr, g_arr, table, out_hbm):
        zero_acc()
        plsc.subcore_barrier()
        scan_pass(s_arr, g_arr, table)
        plsc.subcore_barrier()
        write_out(out_hbm)
        plsc.subcore_barrier()

    pltpu.sync_copy(ones_r, ones_v)
    phase(dst_e, src_e, user_t, sum_i)
    phase(dst_e, None, None, cnt_i)
    phase(src_e, dst_e, item_t, sum_u)
    phase(src_e, None, None, cnt_u)


_sc_call = pl.kernel(
    _sc_body,
    out_type=(
        jax.ShapeDtypeStruct((N_NODE, D), jnp.float32),
        jax.ShapeDtypeStruct((N_NODE, D), jnp.float32),
        jax.ShapeDtypeStruct((N_NODE, D), jnp.float32),
        jax.ShapeDtypeStruct((N_NODE, D), jnp.float32),
    ),
    mesh=plsc.VectorSubcoreMesh(core_axis_name="c", subcore_axis_name="s"),
    compiler_params=pltpu.CompilerParams(use_tc_tiling_on_sc=False),
    scratch_types=(
        pltpu.VMEM((BLK,), jnp.int32),
        pltpu.VMEM((BLK,), jnp.int32),
        pltpu.VMEM((CH,), jnp.int32),
        pltpu.VMEM((CH, D), jnp.float32),
        pltpu.VMEM((CH, D), jnp.float32),
        pltpu.VMEM_SHARED((ACC_ROWS, D), jnp.float32),
        pltpu.SemaphoreType.DMA,
    ),
)

BR = 1000


def _dense_body(hu, su, cu, hi, si, ci, wsT, wnT, bb, ou, oi):
    ws = wsT[...]
    wn = wnT[...]
    bv = bb[...]
    agg_u = su[...] * (1.0 / jnp.maximum(cu[...], 1.0))
    ou[...] = (jnp.dot(hu[...], ws, preferred_element_type=jnp.float32)
               + jnp.dot(agg_u, wn, preferred_element_type=jnp.float32) + bv)
    agg_i = si[...] * (1.0 / jnp.maximum(ci[...], 1.0))
    oi[...] = (jnp.dot(hi[...], ws, preferred_element_type=jnp.float32)
               + jnp.dot(agg_i, wn, preferred_element_type=jnp.float32) + bv)


def _row_spec():
    return pl.BlockSpec((BR, D), lambda i: (i, 0))


def _full_spec(shape):
    return pl.BlockSpec(shape, lambda i: tuple(0 for _ in shape))


_dense_call = pl.pallas_call(
    _dense_body,
    grid=(N_NODE // BR,),
    in_specs=[
        _row_spec(), _row_spec(), _row_spec(),
        _row_spec(), _row_spec(), _row_spec(),
        _full_spec((D, D)), _full_spec((D, D)), _full_spec((1, D)),
    ],
    out_specs=[_row_spec(), _row_spec()],
    out_shape=(
        jax.ShapeDtypeStruct((N_NODE, D), jnp.float32),
        jax.ShapeDtypeStruct((N_NODE, D), jnp.float32),
    ),
)


def kernel(edge_index, user_table, item_table, W_self, W_neigh, b):
    edge_index = edge_index.astype(jnp.int32)
    zrow = jnp.zeros((3136, D), jnp.float32)
    ones_r = jnp.ones((CH, D), jnp.float32)
    sum_i, cnt_i, sum_u, cnt_u = _sc_call(
        edge_index[0], edge_index[1], user_table, item_table, zrow, ones_r)
    out_user, out_item = _dense_call(
        user_table, sum_u, cnt_u, item_table, sum_i, cnt_i,
        W_self.T, W_neigh.T, b.reshape(1, D))
    return (out_user, out_item)


# speedup vs baseline: 3.7508x; 1.0943x over previous
import jax
import jax.numpy as jnp
from jax import lax
from jax.experimental import pallas as pl
from jax.experimental.pallas import tpu as pltpu
from jax.experimental.pallas import tpu_sc as plsc

N_NODE = 100000
D = 32
E = 1600000
HALF = 50000
ACC_ROWS = 50176
DUMMY = HALF
EPT = E // 16
BLK = 2000
NBLK = EPT // BLK
CH = 80
NCH = BLK // CH


def _sc_body(src_e, dst_e, user_t, item_t, zrow, ones_r,
             sum_i, cnt_i, sum_u, cnt_u,
             gbuf, dbuf, sidx, rows, ones_v, acc, gsem, ssem):
    c = lax.axis_index("c")
    s = lax.axis_index("s")
    lo = c * HALF

    def zero_acc():
        pltpu.sync_copy(zrow, acc.at[pl.ds(s * 3136, 3136)])

    def scan_pass(s_arr, g_arr, table):
        @pl.loop(0, NBLK)
        def blk_body(blk):
            base = s * EPT + blk * BLK
            if table is not None:
                pltpu.sync_copy(g_arr.at[pl.ds(base, BLK)], gbuf)
            pltpu.sync_copy(s_arr.at[pl.ds(base, BLK)], dbuf)

            @pl.loop(0, NCH)
            def scan_ch(t):
                for q in range(CH // 16):
                    dv = dbuf[pl.ds(t * CH + q * 16, 16)]
                    ld = dv - lo
                    m = (ld >= 0) & (ld < HALF)
                    ld = jnp.where(m, ld, DUMMY)
                    sidx[t, pl.ds(q * 16, 16)] = ld

            if table is not None:
                def g_desc(t, slot):
                    return pltpu.make_async_copy(
                        table.at[gbuf.at[pl.ds(t * CH, CH)]],
                        rows.at[slot], gsem.at[slot])

                def s_desc(t, slot):
                    return pltpu.make_async_copy(
                        rows.at[slot], acc.at[sidx.at[t]], ssem.at[slot])

                for t in range(4):
                    g_desc(t, t).start()

                @pl.loop(0, NCH)
                def main_ch(t):
                    g_desc(t, t % 8).wait()
                    pltpu.async_copy(rows.at[t % 8], acc.at[sidx.at[t]],
                                     ssem.at[t % 8], add=True)
                    u = t + 4

                    @pl.when(u < NCH)
                    def _():
                        @pl.when(t >= 4)
                        def _():
                            s_desc(t - 4, u % 8).wait()
                        g_desc(u, u % 8).start()

                for k in range(NCH - 8, NCH):
                    s_desc(k, k % 8).wait()
            else:
                def o_desc(t, slot):
                    return pltpu.make_async_copy(
                        ones_v, acc.at[sidx.at[t]], ssem.at[slot])

                @pl.loop(0, NCH)
                def ones_ch(t):
                    @pl.when(t >= 8)
                    def _():
                        o_desc(t - 8, t % 8).wait()
                    pltpu.async_copy(ones_v, acc.at[sidx.at[t]],
                                     ssem.at[t % 8], add=True)

                for k in range(NCH - 8, NCH):
                    o_desc(k, k % 8).wait()

    def write_out(out_hbm):
        @pl.loop(0, 40)
        def wrs(k):
            j = k * 16 + s

            @pl.when(j < HALF // 80)
            def _():
                pltpu.sync_copy(acc.at[pl.ds(j * 80, 80)],
                                out_hbm.at[pl.ds(c * HALF + j * 80, 80)])

    def phase(s_arr, g_arr, table, out_hbm):
        zero_acc()
        plsc.subcore_barrier()
        scan_pass(s_arr, g_arr, table)
        plsc.subcore_barrier()
        write_out(out_hbm)
        plsc.subcore_barrier()

    pltpu.sync_copy(ones_r, ones_v)
    phase(dst_e, src_e, user_t, sum_i)
    phase(dst_e, None, None, cnt_i)
    phase(src_e, dst_e, item_t, sum_u)
    phase(src_e, None, None, cnt_u)


_sc_call = pl.kernel(
    _sc_body,
    out_type=(
        jax.ShapeDtypeStruct((N_NODE, D), jnp.float32),
        jax.ShapeDtypeStruct((N_NODE, D), jnp.float32),
        jax.ShapeDtypeStruct((N_NODE, D), jnp.float32),
        jax.ShapeDtypeStruct((N_NODE, D), jnp.float32),
    ),
    mesh=plsc.VectorSubcoreMesh(core_axis_name="c", subcore_axis_name="s"),
    compiler_params=pltpu.CompilerParams(use_tc_tiling_on_sc=False),
    scratch_types=(
        pltpu.VMEM((BLK,), jnp.int32),
        pltpu.VMEM((BLK,), jnp.int32),
        pltpu.VMEM((NCH, CH), jnp.int32),
        pltpu.VMEM((8, CH, D), jnp.float32),
        pltpu.VMEM((CH, D), jnp.float32),
        pltpu.VMEM_SHARED((ACC_ROWS, D), jnp.float32),
        pltpu.SemaphoreType.DMA((8,)),
        pltpu.SemaphoreType.DMA((8,)),
    ),
)

BR = 1000


def _dense_body(hu, su, cu, hi, si, ci, wsT, wnT, bb, ou, oi):
    ws = wsT[...]
    wn = wnT[...]
    bv = bb[...]
    agg_u = su[...] * (1.0 / jnp.maximum(cu[...], 1.0))
    ou[...] = (jnp.dot(hu[...], ws, preferred_element_type=jnp.float32)
               + jnp.dot(agg_u, wn, preferred_element_type=jnp.float32) + bv)
    agg_i = si[...] * (1.0 / jnp.maximum(ci[...], 1.0))
    oi[...] = (jnp.dot(hi[...], ws, preferred_element_type=jnp.float32)
               + jnp.dot(agg_i, wn, preferred_element_type=jnp.float32) + bv)


def _row_spec():
    return pl.BlockSpec((BR, D), lambda i: (i, 0))


def _full_spec(shape):
    return pl.BlockSpec(shape, lambda i: tuple(0 for _ in shape))


_dense_call = pl.pallas_call(
    _dense_body,
    grid=(N_NODE // BR,),
    in_specs=[
        _row_spec(), _row_spec(), _row_spec(),
        _row_spec(), _row_spec(), _row_spec(),
        _full_spec((D, D)), _full_spec((D, D)), _full_spec((1, D)),
    ],
    out_specs=[_row_spec(), _row_spec()],
    out_shape=(
        jax.ShapeDtypeStruct((N_NODE, D), jnp.float32),
        jax.ShapeDtypeStruct((N_NODE, D), jnp.float32),
    ),
)


def kernel(edge_index, user_table, item_table, W_self, W_neigh, b):
    edge_index = edge_index.astype(jnp.int32)
    zrow = jnp.zeros((3136, D), jnp.float32)
    ones_r = jnp.ones((CH, D), jnp.float32)
    sum_i, cnt_i, sum_u, cnt_u = _sc_call(
        edge_index[0], edge_index[1], user_table, item_table, zrow, ones_r)
    out_user, out_item = _dense_call(
        user_table, sum_u, cnt_u, item_table, sum_i, cnt_i,
        W_self.T, W_neigh.T, b.reshape(1, D))
    return (out_user, out_item)
